# baseline (speedup 1.0000x reference)
import functools

import jax
import jax.numpy as jnp
from jax import lax
from jax.experimental import pallas as pl
from jax.experimental.pallas import tpu as pltpu
from jax.experimental.pallas import tpu_sc as plsc

_H = 64
_NC = 2
_NS = 16
_NW = _NC * _NS
_CHUNK = 512


def _proj_body(emb_ref, w_ref, b_ref, out_ref):
    out_ref[...] = lax.dot_general(
        emb_ref[...], w_ref[...], (((1,), (1,)), ((), ())),
        preferred_element_type=jnp.float32) + b_ref[...]


def _make_sc_call(n_rows: int):
    rows_per_w = n_rows // _NW
    n_chunks = rows_per_w // _CHUNK

    def _sc_body(p_hbm, x_hbm, t_hbm, out_hbm, p_v, t_v, buf):
        wid = lax.axis_index("s") * _NC + lax.axis_index("c")
        base = wid * rows_per_w
        pltpu.sync_copy(p_hbm, p_v)
        lane = lax.iota(jnp.int32, 16)

        def chunk_body(g, carry):
            start = base + g * _CHUNK
            pltpu.sync_copy(t_hbm.at[pl.ds(start, _CHUNK)], t_v)
            pltpu.sync_copy(x_hbm.at[pl.ds(start, _CHUNK)], buf)

            def group_body(i, c2):
                tvec = t_v[pl.ds(i * 16, 16)]
                rvec = i * 16 + lane
                for c in range(_H):
                    cvec = jnp.full((16,), c, jnp.int32)
                    vals = plsc.load_gather(p_v, [tvec, cvec])
                    plsc.addupdate_scatter(buf, [rvec, cvec], vals)
                return c2

            lax.fori_loop(0, _CHUNK // 16, group_body, 0)
            pltpu.sync_copy(buf, out_hbm.at[pl.ds(start, _CHUNK)])
            return carry

        lax.fori_loop(0, n_chunks, chunk_body, 0)

    return pl.kernel(
        _sc_body,
        out_type=jax.ShapeDtypeStruct((n_rows, _H), jnp.float32),
        mesh=plsc.VectorSubcoreMesh(core_axis_name="c", subcore_axis_name="s"),
        scratch_types=[
            pltpu.VMEM((_H, _H), jnp.float32),
            pltpu.VMEM((_CHUNK,), jnp.int32),
            pltpu.VMEM((_CHUNK, _H), jnp.float32),
        ],
        compiler_params=pltpu.CompilerParams(needs_layout_passes=False),
    )


def kernel(x, t, emb_table, W, b):
    batch, hist, h = x.shape
    n_rows = batch * hist
    p = pl.pallas_call(
        _proj_body,
        out_shape=jax.ShapeDtypeStruct((_H, _H), jnp.float32),
    )(emb_table, W, b.reshape(1, _H))
    out = _make_sc_call(n_rows)(
        p, x.reshape(n_rows, h), t.reshape(n_rows))
    return out.reshape(x.shape)


# speedup vs baseline: 2.2155x; 2.2155x over previous
import functools

import jax
import jax.numpy as jnp
from jax import lax
from jax.experimental import pallas as pl
from jax.experimental.pallas import tpu as pltpu
from jax.experimental.pallas import tpu_sc as plsc

_H = 64
_NC = 2
_NS = 16
_NW = _NC * _NS
_CHUNK = 512


def _proj_body(emb_ref, w_ref, b_ref, out_ref):
    out_ref[...] = lax.dot_general(
        emb_ref[...], w_ref[...], (((1,), (1,)), ((), ())),
        preferred_element_type=jnp.float32) + b_ref[...]


def _make_sc_call(n_rows: int):
    rows_per_w = n_rows // _NW
    n_chunks = rows_per_w // _CHUNK

    def _sc_body(p_hbm, x_hbm, t_hbm, out_hbm, p_v, t_v, buf):
        wid = lax.axis_index("s") * _NC + lax.axis_index("c")
        base = wid * rows_per_w
        pltpu.sync_copy(p_hbm, p_v)

        def chunk_body(g, carry):
            start = base + g * _CHUNK
            pltpu.sync_copy(t_hbm.at[pl.ds(start, _CHUNK)], t_v)
            pltpu.sync_copy(x_hbm.at[pl.ds(start, _CHUNK)], buf)

            @plsc.parallel_loop(0, _CHUNK, 16)
            def _rows(i):
                tvec = t_v[pl.ds(i, 16)]
                for h in range(2):
                    vals = []
                    for k in range(8 * h, 8 * h + 8):
                        ti = tvec[k]
                        vals.append([p_v[ti, pl.ds(cg * 16, 16)]
                                     for cg in range(_H // 16)])
                    for k in range(8 * h, 8 * h + 8):
                        for cg in range(_H // 16):
                            plsc.addupdate(buf.at[i + k, pl.ds(cg * 16, 16)],
                                           vals[k - 8 * h][cg])
            pltpu.sync_copy(buf, out_hbm.at[pl.ds(start, _CHUNK)])
            return carry

        lax.fori_loop(0, n_chunks, chunk_body, 0)

    return pl.kernel(
        _sc_body,
        out_type=jax.ShapeDtypeStruct((n_rows, _H), jnp.float32),
        mesh=plsc.VectorSubcoreMesh(core_axis_name="c", subcore_axis_name="s"),
        scratch_types=[
            pltpu.VMEM((_H, _H), jnp.float32),
            pltpu.VMEM((_CHUNK,), jnp.int32),
            pltpu.VMEM((_CHUNK, _H), jnp.float32),
        ],
    )


def kernel(x, t, emb_table, W, b):
    batch, hist, h = x.shape
    n_rows = batch * hist
    p = pl.pallas_call(
        _proj_body,
        out_shape=jax.ShapeDtypeStruct((_H, _H), jnp.float32),
    )(emb_table, W, b.reshape(1, _H))
    out = _make_sc_call(n_rows)(
        p, x.reshape(n_rows, h), t.reshape(n_rows))
    return out.reshape(x.shape)


# speedup vs baseline: 2.4374x; 1.1002x over previous
import functools

import jax
import jax.numpy as jnp
from jax import lax
from jax.experimental import pallas as pl
from jax.experimental.pallas import tpu as pltpu
from jax.experimental.pallas import tpu_sc as plsc

_H = 64
_NC = 2
_NS = 16
_NW = _NC * _NS
_CHUNK = 400


def _proj_body(emb_ref, w_ref, b_ref, out_ref):
    out_ref[...] = lax.dot_general(
        emb_ref[...], w_ref[...], (((1,), (1,)), ((), ())),
        preferred_element_type=jnp.float32) + b_ref[...]


def _make_sc_call(n_rows: int):
    rows_per_w = n_rows // _NW
    n_chunks = rows_per_w // _CHUNK
    assert n_chunks % 2 == 0

    def _sc_body(p_hbm, x_hbm, t_hbm, out_hbm,
                 p_v, t0, t1, buf0, buf1, isem0, isem1, osem0, osem1):
        wid = lax.axis_index("s") * _NC + lax.axis_index("c")
        base = wid * rows_per_w
        pltpu.sync_copy(p_hbm, p_v)
        slots = ((t0, buf0, isem0, osem0), (t1, buf1, isem1, osem1))

        def start_in(g, sl):
            start = base + g * _CHUNK
            t_v, buf, isem, _ = slots[sl]
            pltpu.async_copy(t_hbm.at[pl.ds(start, _CHUNK)], t_v, isem)
            pltpu.async_copy(x_hbm.at[pl.ds(start, _CHUNK)], buf, isem)

        def wait_in(sl):
            t_v, buf, isem, _ = slots[sl]
            pltpu.make_async_copy(t_hbm.at[pl.ds(0, _CHUNK)], t_v, isem).wait()
            pltpu.make_async_copy(x_hbm.at[pl.ds(0, _CHUNK)], buf, isem).wait()

        def start_out(g, sl):
            start = base + g * _CHUNK
            _, buf, _, osem = slots[sl]
            pltpu.async_copy(buf, out_hbm.at[pl.ds(start, _CHUNK)], osem)

        def wait_out(sl):
            _, buf, _, osem = slots[sl]
            pltpu.make_async_copy(
                buf, out_hbm.at[pl.ds(0, _CHUNK)], osem).wait()

        def compute(sl):
            t_v, buf, _, _ = slots[sl]

            @plsc.parallel_loop(0, _CHUNK, 16)
            def _rows(i):
                tvec = t_v[pl.ds(i, 16)]
                for h in range(2):
                    vals = []
                    for k in range(8 * h, 8 * h + 8):
                        ti = tvec[k]
                        vals.append([p_v[ti, pl.ds(cg * 16, 16)]
                                     for cg in range(_H // 16)])
                    for k in range(8 * h, 8 * h + 8):
                        for cg in range(_H // 16):
                            plsc.addupdate(
                                buf.at[i + k, pl.ds(cg * 16, 16)],
                                vals[k - 8 * h][cg])

        start_in(0, 0)
        start_in(1, 1)

        def pair_body(gp, carry):
            g = gp * 2
            for sl in range(2):
                wait_in(sl)
                compute(sl)
                start_out(g + sl, sl)

                @pl.when(gp + 1 < n_chunks // 2)
                def _():
                    wait_out(sl)
                    start_in(g + sl + 2, sl)

            return carry

        lax.fori_loop(0, n_chunks // 2, pair_body, 0)
        wait_out(0)
        wait_out(1)

    return pl.kernel(
        _sc_body,
        out_type=jax.ShapeDtypeStruct((n_rows, _H), jnp.float32),
        mesh=plsc.VectorSubcoreMesh(core_axis_name="c", subcore_axis_name="s"),
        scratch_types=[
            pltpu.VMEM((_H, _H), jnp.float32),
            pltpu.VMEM((_CHUNK,), jnp.int32),
            pltpu.VMEM((_CHUNK,), jnp.int32),
            pltpu.VMEM((_CHUNK, _H), jnp.float32),
            pltpu.VMEM((_CHUNK, _H), jnp.float32),
            pltpu.SemaphoreType.DMA,
            pltpu.SemaphoreType.DMA,
            pltpu.SemaphoreType.DMA,
            pltpu.SemaphoreType.DMA,
        ],
    )


def kernel(x, t, emb_table, W, b):
    batch, hist, h = x.shape
    n_rows = batch * hist
    p = pl.pallas_call(
        _proj_body,
        out_shape=jax.ShapeDtypeStruct((_H, _H), jnp.float32),
    )(emb_table, W, b.reshape(1, _H))
    out = _make_sc_call(n_rows)(
        p, x.reshape(n_rows, h), t.reshape(n_rows))
    return out.reshape(x.shape)
